# Initial kernel scaffold; baseline (speedup 1.0000x reference)
import jax
import jax.numpy as jnp
from jax.experimental import pallas as pl


def kernel(hidden_states, proto, w_gate, w_up, w_down):
    raise NotImplementedError("write your pallas kernel here")


# speedup vs baseline: 1.7905x; 1.7905x over previous
import functools

import jax
import jax.numpy as jnp
from jax.experimental import pallas as pl
from jax.experimental.pallas import tpu as pltpu

NE = 8
DM = 1024
DF = 512
TOPK = 2
BM = 256


def _moe_block_kernel(x_ref, protoT_ref, wg_ref, wu_ref, wd_ref,
                      out_ref, logits_ref):
    x = x_ref[...]

    xsq = jnp.sum(x * x, axis=1, keepdims=True)
    xnorm = jnp.sqrt(xsq)
    xn = x / jnp.maximum(xnorm, 1e-12)
    pT = protoT_ref[...]
    psq = jnp.sum(pT * pT, axis=0, keepdims=True)
    pn = pT / jnp.maximum(jnp.sqrt(psq), 1e-12)
    logits = jax.lax.dot_general(
        xn.astype(jnp.bfloat16), pn.astype(jnp.bfloat16),
        (((1,), (0,)), ((), ())),
        preferred_element_type=jnp.float32)
    logits_ref[...] = logits

    m = jnp.max(logits, axis=1, keepdims=True)
    ex = jnp.exp(logits - m)
    probs = ex / jnp.sum(ex, axis=1, keepdims=True)
    p1 = jnp.max(probs, axis=1, keepdims=True)
    masked = jnp.where(probs >= p1, -jnp.inf, probs)
    p2 = jnp.max(masked, axis=1, keepdims=True)
    cw = jnp.where(probs >= p2, probs, 0.0)

    xb = x.astype(jnp.bfloat16)
    acc = jnp.zeros((x.shape[0], DM), dtype=jnp.float32)
    for e in range(NE):
        g = jax.lax.dot_general(
            xb, wg_ref[e], (((1,), (0,)), ((), ())),
            preferred_element_type=jnp.float32)
        u = jax.lax.dot_general(
            xb, wu_ref[e], (((1,), (0,)), ((), ())),
            preferred_element_type=jnp.float32)
        h = (g / (1.0 + jnp.exp(-g))) * u
        hb = (h * cw[:, e:e + 1]).astype(jnp.bfloat16)
        acc = acc + jax.lax.dot_general(
            hb, wd_ref[e], (((1,), (0,)), ((), ())),
            preferred_element_type=jnp.float32)
    out_ref[...] = acc


@jax.jit
def kernel(hidden_states, proto, w_gate, w_up, w_down):
    B, S, D = hidden_states.shape
    T = B * S
    x = hidden_states.reshape(T, D)
    protoT = proto.T
    wg = w_gate.astype(jnp.bfloat16)
    wu = w_up.astype(jnp.bfloat16)
    wd = w_down.astype(jnp.bfloat16)

    grid = (T // BM,)
    out, logits = pl.pallas_call(
        _moe_block_kernel,
        grid=grid,
        in_specs=[
            pl.BlockSpec((BM, DM), lambda i: (i, 0)),
            pl.BlockSpec((DM, NE), lambda i: (0, 0)),
            pl.BlockSpec((NE, DM, DF), lambda i: (0, 0, 0)),
            pl.BlockSpec((NE, DM, DF), lambda i: (0, 0, 0)),
            pl.BlockSpec((NE, DF, DM), lambda i: (0, 0, 0)),
        ],
        out_specs=[
            pl.BlockSpec((BM, DM), lambda i: (i, 0)),
            pl.BlockSpec((BM, NE), lambda i: (i, 0)),
        ],
        out_shape=[
            jax.ShapeDtypeStruct((T, DM), jnp.float32),
            jax.ShapeDtypeStruct((T, NE), jnp.float32),
        ],
    )(x, protoT, wg, wu, wd)
    return out.reshape(B, S, D), logits
